# baseline (speedup 1.0000x reference)
import functools

import jax
import jax.numpy as jnp
from jax import lax
from jax.experimental import pallas as pl
from jax.experimental.pallas import tpu as pltpu
from jax.experimental.pallas import tpu_sc as plsc

_NC = 2
_NS = 16
_CHUNK = 112
_NBUF = 3
_G = 8
_LOOK = 2


def _sc_spmm(fea, src2, dst2, ew2, zeros_blk, nc):
    n, d = fea.shape
    piece = 200
    pieces = n // piece
    groups = d // 16
    nseg = nc // _G

    mesh = plsc.VectorSubcoreMesh(core_axis_name="c", subcore_axis_name="s")

    @functools.partial(
        pl.kernel,
        out_type=jax.ShapeDtypeStruct((_NC * n, d), jnp.float32),
        mesh=mesh,
        scratch_types=[
            pltpu.VMEM((2, _G, _CHUNK), jnp.int32),
            pltpu.VMEM((2, _G, _CHUNK), jnp.int32),
            pltpu.VMEM((2, _G, _CHUNK), jnp.float32),
            pltpu.VMEM((_NBUF, _CHUNK, d), jnp.float32),
            pltpu.VMEM_SHARED((n, d), jnp.float32),
            [pltpu.SemaphoreType.DMA] * _NBUF,
            [pltpu.SemaphoreType.DMA] * _NBUF,
            [pltpu.SemaphoreType.DMA] * 2,
        ],
    )
    def spmm(fea_hbm, src_hbm, dst_hbm, ew_hbm, zero_hbm, out_hbm,
             src_v, dst_v, ew_v, rows_v, acc, gsems, ssems, isems):
        cid = lax.axis_index("c")
        sid = lax.axis_index("s")
        wid = sid * _NC + cid
        c0 = wid * nc

        def fire_seg(seg, slot):
            pltpu.async_copy(src_hbm.at[pl.ds(c0 + seg * _G, _G)],
                             src_v.at[slot], isems[slot])
            pltpu.async_copy(dst_hbm.at[pl.ds(c0 + seg * _G, _G)],
                             dst_v.at[slot], isems[slot])
            pltpu.async_copy(ew_hbm.at[pl.ds(c0 + seg * _G, _G)],
                             ew_v.at[slot], isems[slot])

        def wait_seg(slot):
            pltpu.make_async_copy(src_hbm.at[pl.ds(c0, _G)], src_v.at[slot],
                                  isems[slot]).wait()
            pltpu.make_async_copy(dst_hbm.at[pl.ds(c0, _G)], dst_v.at[slot],
                                  isems[slot]).wait()
            pltpu.make_async_copy(ew_hbm.at[pl.ds(c0, _G)], ew_v.at[slot],
                                  isems[slot]).wait()

        fire_seg(0, 0)
        wait_seg(0)
        fire_seg(1, 1)

        my_pieces = (pieces // _NS) + jnp.where(sid < (pieces % _NS), 1, 0)

        def init_body(t, c_):
            off = (sid + t * _NS) * piece
            pltpu.sync_copy(zero_hbm, acc.at[pl.ds(off, piece)])
            return c_

        lax.fori_loop(0, my_pieces, init_body, 0)
        plsc.subcore_barrier()

        def fire_gather(j, b):
            slot, pos = (j // _G) % 2, j % _G
            pltpu.async_copy(fea_hbm.at[src_v.at[slot, pos]], rows_v.at[b],
                             gsems[b])

        def wait_gather(b):
            pltpu.make_async_copy(fea_hbm.at[src_v.at[0, 0]], rows_v.at[b],
                                  gsems[b]).wait()

        def fire_scatter(j, b):
            slot, pos = (j // _G) % 2, j % _G
            pltpu.async_copy(rows_v.at[b], acc.at[dst_v.at[slot, pos]],
                             ssems[b], add=True)

        def wait_scatter(b):
            pltpu.make_async_copy(rows_v.at[b], acc.at[dst_v.at[0, 0]],
                                  ssems[b]).wait()

        def scale(j, b):
            slot, pos = (j // _G) % 2, j % _G

            @plsc.parallel_loop(0, _CHUNK // 16, step=1)
            def _(s):
                wv = ew_v[slot, pos, pl.ds(s * 16, 16)]
                for lane in range(16):
                    w = wv[lane]
                    row = s * 16 + lane
                    for g in range(groups):
                        sl = pl.ds(g * 16, 16)
                        rows_v[b, row, sl] = rows_v[b, row, sl] * w

        for jj in range(_LOOK):
            fire_gather(jj, jj)

        def pipe_body(t, c_):
            for b in range(_NBUF):
                j = t * _NBUF + b
                seg = j // _G
                pos = j % _G

                refill = (pos == 1) & (seg >= 1) & (seg + 1 < nseg)
                for slot in range(2):
                    @pl.when(refill & ((seg + 1) % 2 == slot))
                    def _(slot=slot):
                        fire_seg(seg + 1, slot)

                wait_gather(b)
                scale(j, b)
                fire_scatter(j, b)

                @pl.when(j >= 1)
                def _():
                    wait_scatter((b + _NBUF - 1) % _NBUF)

                segwait = ((pos == (_G - _LOOK) % _G)
                           & ((j + _LOOK) // _G == seg + 1)
                           & (seg + 1 < nseg))
                for slot in range(2):
                    @pl.when(segwait & ((seg + 1) % 2 == slot))
                    def _(slot=slot):
                        wait_seg(slot)

                @pl.when(j + _LOOK < nc)
                def _():
                    fire_gather(j + _LOOK, (b + _LOOK) % _NBUF)

            return c_

        lax.fori_loop(0, nc // _NBUF, pipe_body, 0)
        wait_scatter((nc - 1) % _NBUF)
        plsc.subcore_barrier()

        def out_body(t, c_):
            off = (sid + t * _NS) * piece
            pltpu.sync_copy(acc.at[pl.ds(off, piece)],
                            out_hbm.at[pl.ds(cid * n + off, piece)])
            return c_

        lax.fori_loop(0, my_pieces, out_body, 0)

    return spmm(fea, src2, dst2, ew2, zeros_blk)


def _mm_body(p0_ref, p1_ref, w_ref, b_ref, o_ref):
    s = p0_ref[...] + p1_ref[...]
    o_ref[...] = jnp.dot(s, w_ref[...], preferred_element_type=jnp.float32) + b_ref[...]


def _tc_matmul_bias(parts, w, b):
    n2, d_in = parts.shape
    n = n2 // _NC
    d_out = w.shape[1]
    blk = 1000
    nb = n // blk
    return pl.pallas_call(
        _mm_body,
        grid=(nb,),
        in_specs=[
            pl.BlockSpec((blk, d_in), lambda i: (i, 0)),
            pl.BlockSpec((blk, d_in), lambda i: (i + nb, 0)),
            pl.BlockSpec((d_in, d_out), lambda i: (0, 0)),
            pl.BlockSpec((1, d_out), lambda i: (0, 0)),
        ],
        out_specs=pl.BlockSpec((blk, d_out), lambda i: (i, 0)),
        out_shape=jax.ShapeDtypeStruct((n, d_out), jnp.float32),
    )(parts, parts, w, b.reshape(1, d_out))


def kernel(fea, edge_index, edge_weight, W, b):
    src = edge_index[0].astype(jnp.int32)
    dst = edge_index[1].astype(jnp.int32)
    n, d = fea.shape
    e = src.shape[0]
    nw = _NC * _NS
    unit = _NBUF * _G * _CHUNK
    per_tile = -(-e // (nw * unit)) * unit
    e_pad = nw * per_tile
    nc = per_tile // _CHUNK
    pad = e_pad - e
    fill = (jnp.arange(pad, dtype=jnp.int32) * 7) % jnp.int32(n)
    src2 = jnp.concatenate([src, fill]).reshape(-1, _CHUNK)
    dst2 = jnp.concatenate([dst, fill]).reshape(-1, _CHUNK)
    ew2 = jnp.concatenate([edge_weight.astype(jnp.float32),
                           jnp.zeros((pad,), jnp.float32)]).reshape(-1, _CHUNK)
    zeros_blk = jnp.zeros((200, d), jnp.float32)
    parts = _sc_spmm(fea, src2, dst2, ew2, zeros_blk, nc)
    return _tc_matmul_bias(parts, W, b)


# speedup vs baseline: 1.0803x; 1.0803x over previous
import functools

import jax
import jax.numpy as jnp
from jax import lax
from jax.experimental import pallas as pl
from jax.experimental.pallas import tpu as pltpu
from jax.experimental.pallas import tpu_sc as plsc

_NC = 2
_NS = 16
_CHUNK = 112
_NBUF = 3
_G = 8
_LOOK = 2


def _sc_spmm(fea, src2, dst2, ew2, zeros_blk, nc):
    n, d = fea.shape
    piece = 200
    pieces = n // piece
    groups = d // 16
    nseg = nc // _G

    mesh = plsc.VectorSubcoreMesh(core_axis_name="c", subcore_axis_name="s")

    @functools.partial(
        pl.kernel,
        out_type=jax.ShapeDtypeStruct((_NC * n, d), jnp.float32),
        mesh=mesh,
        scratch_types=[
            pltpu.VMEM((2, _G, _CHUNK), jnp.int32),
            pltpu.VMEM((2, _G, _CHUNK), jnp.int32),
            pltpu.VMEM((2, _G, _CHUNK), jnp.float32),
            pltpu.VMEM((_NBUF, _CHUNK, d), jnp.float32),
            pltpu.VMEM_SHARED((n, d), jnp.float32),
            [pltpu.SemaphoreType.DMA] * _NBUF,
            [pltpu.SemaphoreType.DMA] * _NBUF,
            [pltpu.SemaphoreType.DMA] * 2,
        ],
    )
    def spmm(fea_hbm, src_hbm, dst_hbm, ew_hbm, zero_hbm, out_hbm,
             src_v, dst_v, ew_v, rows_v, acc, gsems, ssems, isems):
        cid = lax.axis_index("c")
        sid = lax.axis_index("s")
        wid = sid * _NC + cid
        c0 = wid * nc

        def fire_seg(seg, slot):
            pltpu.async_copy(src_hbm.at[pl.ds(c0 + seg * _G, _G)],
                             src_v.at[slot], isems[slot])
            pltpu.async_copy(dst_hbm.at[pl.ds(c0 + seg * _G, _G)],
                             dst_v.at[slot], isems[slot])
            pltpu.async_copy(ew_hbm.at[pl.ds(c0 + seg * _G, _G)],
                             ew_v.at[slot], isems[slot])

        def wait_seg(slot):
            pltpu.make_async_copy(src_hbm.at[pl.ds(c0, _G)], src_v.at[slot],
                                  isems[slot]).wait()
            pltpu.make_async_copy(dst_hbm.at[pl.ds(c0, _G)], dst_v.at[slot],
                                  isems[slot]).wait()
            pltpu.make_async_copy(ew_hbm.at[pl.ds(c0, _G)], ew_v.at[slot],
                                  isems[slot]).wait()

        fire_seg(0, 0)
        wait_seg(0)
        fire_seg(1, 1)

        my_pieces = (pieces // _NS) + jnp.where(sid < (pieces % _NS), 1, 0)

        def init_body(t, c_):
            off = (sid + t * _NS) * piece
            pltpu.sync_copy(zero_hbm, acc.at[pl.ds(off, piece)])
            return c_

        lax.fori_loop(0, my_pieces, init_body, 0)
        plsc.subcore_barrier()

        def fire_gather(j, b):
            slot, pos = (j // _G) % 2, j % _G
            pltpu.async_copy(fea_hbm.at[src_v.at[slot, pos]], rows_v.at[b],
                             gsems[b])

        def wait_gather(b):
            pltpu.make_async_copy(fea_hbm.at[src_v.at[0, 0]], rows_v.at[b],
                                  gsems[b]).wait()

        def fire_scatter(j, b):
            slot, pos = (j // _G) % 2, j % _G
            pltpu.async_copy(rows_v.at[b], acc.at[dst_v.at[slot, pos]],
                             ssems[b], add=True)

        def wait_scatter(b):
            pltpu.make_async_copy(rows_v.at[b], acc.at[dst_v.at[0, 0]],
                                  ssems[b]).wait()

        def scale(j, b):
            slot, pos = (j // _G) % 2, j % _G

            def scale_body(s, c_):
                wv = ew_v[slot, pos, pl.ds(s * 16, 16)]
                for lane in range(16):
                    w = wv[lane]
                    row = s * 16 + lane
                    for g in range(groups):
                        sl = pl.ds(g * 16, 16)
                        rows_v[b, row, sl] = rows_v[b, row, sl] * w
                return c_

            lax.fori_loop(0, _CHUNK // 16, scale_body, 0, unroll=2)

        for jj in range(_LOOK):
            fire_gather(jj, jj)

        def pipe_body(t, c_):
            for b in range(_NBUF):
                j = t * _NBUF + b
                seg = j // _G
                pos = j % _G

                refill = (pos == 1) & (seg >= 1) & (seg + 1 < nseg)
                for slot in range(2):
                    @pl.when(refill & ((seg + 1) % 2 == slot))
                    def _(slot=slot):
                        fire_seg(seg + 1, slot)

                wait_gather(b)
                scale(j, b)
                fire_scatter(j, b)

                @pl.when(j >= 1)
                def _():
                    wait_scatter((b + _NBUF - 1) % _NBUF)

                segwait = ((pos == (_G - _LOOK) % _G)
                           & ((j + _LOOK) // _G == seg + 1)
                           & (seg + 1 < nseg))
                for slot in range(2):
                    @pl.when(segwait & ((seg + 1) % 2 == slot))
                    def _(slot=slot):
                        wait_seg(slot)

                @pl.when(j + _LOOK < nc)
                def _():
                    fire_gather(j + _LOOK, (b + _LOOK) % _NBUF)

            return c_

        lax.fori_loop(0, nc // _NBUF, pipe_body, 0)
        wait_scatter((nc - 1) % _NBUF)
        plsc.subcore_barrier()

        def out_body(t, c_):
            off = (sid + t * _NS) * piece
            pltpu.sync_copy(acc.at[pl.ds(off, piece)],
                            out_hbm.at[pl.ds(cid * n + off, piece)])
            return c_

        lax.fori_loop(0, my_pieces, out_body, 0)

    return spmm(fea, src2, dst2, ew2, zeros_blk)


def _mm_body(p0_ref, p1_ref, w_ref, b_ref, o_ref):
    s = p0_ref[...] + p1_ref[...]
    o_ref[...] = jnp.dot(s, w_ref[...], preferred_element_type=jnp.float32) + b_ref[...]


def _tc_matmul_bias(parts, w, b):
    n2, d_in = parts.shape
    n = n2 // _NC
    d_out = w.shape[1]
    blk = 1000
    nb = n // blk
    return pl.pallas_call(
        _mm_body,
        grid=(nb,),
        in_specs=[
            pl.BlockSpec((blk, d_in), lambda i: (i, 0)),
            pl.BlockSpec((blk, d_in), lambda i: (i + nb, 0)),
            pl.BlockSpec((d_in, d_out), lambda i: (0, 0)),
            pl.BlockSpec((1, d_out), lambda i: (0, 0)),
        ],
        out_specs=pl.BlockSpec((blk, d_out), lambda i: (i, 0)),
        out_shape=jax.ShapeDtypeStruct((n, d_out), jnp.float32),
    )(parts, parts, w, b.reshape(1, d_out))


def kernel(fea, edge_index, edge_weight, W, b):
    src = edge_index[0].astype(jnp.int32)
    dst = edge_index[1].astype(jnp.int32)
    n, d = fea.shape
    e = src.shape[0]
    nw = _NC * _NS
    unit = _NBUF * _G * _CHUNK
    per_tile = -(-e // (nw * unit)) * unit
    e_pad = nw * per_tile
    nc = per_tile // _CHUNK
    pad = e_pad - e
    fill = (jnp.arange(pad, dtype=jnp.int32) * 7) % jnp.int32(n)
    src2 = jnp.concatenate([src, fill]).reshape(-1, _CHUNK)
    dst2 = jnp.concatenate([dst, fill]).reshape(-1, _CHUNK)
    ew2 = jnp.concatenate([edge_weight.astype(jnp.float32),
                           jnp.zeros((pad,), jnp.float32)]).reshape(-1, _CHUNK)
    zeros_blk = jnp.zeros((200, d), jnp.float32)
    parts = _sc_spmm(fea, src2, dst2, ew2, zeros_blk, nc)
    return _tc_matmul_bias(parts, W, b)


# speedup vs baseline: 1.0858x; 1.0050x over previous
import functools

import jax
import jax.numpy as jnp
from jax import lax
from jax.experimental import pallas as pl
from jax.experimental.pallas import tpu as pltpu
from jax.experimental.pallas import tpu_sc as plsc

_NC = 2
_NS = 16
_CHUNK = 112
_NBUF = 3
_G = 8
_LOOK = 2


def _sc_spmm(fea, src2, dst2, ew2, zeros_blk, nc):
    n, d = fea.shape
    piece = 200
    pieces = n // piece
    groups = d // 16
    nseg = nc // _G

    mesh = plsc.VectorSubcoreMesh(core_axis_name="c", subcore_axis_name="s")

    @functools.partial(
        pl.kernel,
        out_type=jax.ShapeDtypeStruct((_NC * n, d), jnp.float32),
        mesh=mesh,
        scratch_types=[
            pltpu.VMEM((2, _G, _CHUNK), jnp.int32),
            pltpu.VMEM((2, _G, _CHUNK), jnp.int32),
            pltpu.VMEM((2, _G, _CHUNK), jnp.float32),
            pltpu.VMEM((_NBUF, _CHUNK, d), jnp.float32),
            pltpu.VMEM_SHARED((n, d), jnp.float32),
            [pltpu.SemaphoreType.DMA] * _NBUF,
            [pltpu.SemaphoreType.DMA] * _NBUF,
            [pltpu.SemaphoreType.DMA] * 2,
        ],
    )
    def spmm(fea_hbm, src_hbm, dst_hbm, ew_hbm, zero_hbm, out_hbm,
             src_v, dst_v, ew_v, rows_v, acc, gsems, ssems, isems):
        cid = lax.axis_index("c")
        sid = lax.axis_index("s")
        wid = sid * _NC + cid
        c0 = wid * nc

        def fire_seg(seg, slot):
            pltpu.async_copy(src_hbm.at[pl.ds(c0 + seg * _G, _G)],
                             src_v.at[slot], isems[slot])
            pltpu.async_copy(dst_hbm.at[pl.ds(c0 + seg * _G, _G)],
                             dst_v.at[slot], isems[slot])
            pltpu.async_copy(ew_hbm.at[pl.ds(c0 + seg * _G, _G)],
                             ew_v.at[slot], isems[slot])

        def wait_seg(slot):
            pltpu.make_async_copy(src_hbm.at[pl.ds(c0, _G)], src_v.at[slot],
                                  isems[slot]).wait()
            pltpu.make_async_copy(dst_hbm.at[pl.ds(c0, _G)], dst_v.at[slot],
                                  isems[slot]).wait()
            pltpu.make_async_copy(ew_hbm.at[pl.ds(c0, _G)], ew_v.at[slot],
                                  isems[slot]).wait()

        fire_seg(0, 0)
        wait_seg(0)
        fire_seg(1, 1)

        my_pieces = (pieces // _NS) + jnp.where(sid < (pieces % _NS), 1, 0)

        def init_body(t, c_):
            off = (sid + t * _NS) * piece
            pltpu.sync_copy(zero_hbm, acc.at[pl.ds(off, piece)])
            return c_

        lax.fori_loop(0, my_pieces, init_body, 0)
        plsc.subcore_barrier()

        def fire_gather(j, b):
            slot, pos = (j // _G) % 2, j % _G
            pltpu.async_copy(fea_hbm.at[src_v.at[slot, pos]], rows_v.at[b],
                             gsems[b])

        def wait_gather(b):
            pltpu.make_async_copy(fea_hbm.at[src_v.at[0, 0]], rows_v.at[b],
                                  gsems[b]).wait()

        def fire_scatter(j, b):
            slot, pos = (j // _G) % 2, j % _G
            pltpu.async_copy(rows_v.at[b], acc.at[dst_v.at[slot, pos]],
                             ssems[b], add=True)

        def wait_scatter(b):
            pltpu.make_async_copy(rows_v.at[b], acc.at[dst_v.at[0, 0]],
                                  ssems[b]).wait()

        def scale(j, b):
            slot, pos = (j // _G) % 2, j % _G

            def scale_body(s, c_):
                wv = ew_v[slot, pos, pl.ds(s * 16, 16)]
                for lane in range(16):
                    w = wv[lane]
                    row = s * 16 + lane
                    for g in range(groups):
                        sl = pl.ds(g * 16, 16)
                        rows_v[b, row, sl] = rows_v[b, row, sl] * w
                return c_

            lax.fori_loop(0, _CHUNK // 16, scale_body, 0)

        for jj in range(_LOOK):
            fire_gather(jj, jj)

        def pipe_body(t, c_):
            for b in range(_NBUF):
                j = t * _NBUF + b
                seg = j // _G
                pos = j % _G

                refill = (pos == 1) & (seg >= 1) & (seg + 1 < nseg)
                for slot in range(2):
                    @pl.when(refill & ((seg + 1) % 2 == slot))
                    def _(slot=slot):
                        fire_seg(seg + 1, slot)

                wait_gather(b)
                scale(j, b)
                fire_scatter(j, b)

                @pl.when(j >= 1)
                def _():
                    wait_scatter((b + _NBUF - 1) % _NBUF)

                segwait = ((pos == (_G - _LOOK) % _G)
                           & ((j + _LOOK) // _G == seg + 1)
                           & (seg + 1 < nseg))
                for slot in range(2):
                    @pl.when(segwait & ((seg + 1) % 2 == slot))
                    def _(slot=slot):
                        wait_seg(slot)

                @pl.when(j + _LOOK < nc)
                def _():
                    fire_gather(j + _LOOK, (b + _LOOK) % _NBUF)

            return c_

        lax.fori_loop(0, nc // _NBUF, pipe_body, 0)
        wait_scatter((nc - 1) % _NBUF)
        plsc.subcore_barrier()

        def out_body(t, c_):
            off = (sid + t * _NS) * piece
            pltpu.sync_copy(acc.at[pl.ds(off, piece)],
                            out_hbm.at[pl.ds(cid * n + off, piece)])
            return c_

        lax.fori_loop(0, my_pieces, out_body, 0)

    return spmm(fea, src2, dst2, ew2, zeros_blk)


def _mm_body(p0_ref, p1_ref, w_ref, b_ref, o_ref):
    s = p0_ref[...] + p1_ref[...]
    o_ref[...] = jnp.dot(s, w_ref[...], preferred_element_type=jnp.float32) + b_ref[...]


def _tc_matmul_bias(parts, w, b):
    n2, d_in = parts.shape
    n = n2 // _NC
    d_out = w.shape[1]
    blk = 1000
    nb = n // blk
    return pl.pallas_call(
        _mm_body,
        grid=(nb,),
        in_specs=[
            pl.BlockSpec((blk, d_in), lambda i: (i, 0)),
            pl.BlockSpec((blk, d_in), lambda i: (i + nb, 0)),
            pl.BlockSpec((d_in, d_out), lambda i: (0, 0)),
            pl.BlockSpec((1, d_out), lambda i: (0, 0)),
        ],
        out_specs=pl.BlockSpec((blk, d_out), lambda i: (i, 0)),
        out_shape=jax.ShapeDtypeStruct((n, d_out), jnp.float32),
    )(parts, parts, w, b.reshape(1, d_out))


def kernel(fea, edge_index, edge_weight, W, b):
    src = edge_index[0].astype(jnp.int32)
    dst = edge_index[1].astype(jnp.int32)
    n, d = fea.shape
    e = src.shape[0]
    nw = _NC * _NS
    unit = _NBUF * _G * _CHUNK
    per_tile = -(-e // (nw * unit)) * unit
    e_pad = nw * per_tile
    nc = per_tile // _CHUNK
    pad = e_pad - e
    fill = (jnp.arange(pad, dtype=jnp.int32) * 7) % jnp.int32(n)
    src2 = jnp.concatenate([src, fill]).reshape(-1, _CHUNK)
    dst2 = jnp.concatenate([dst, fill]).reshape(-1, _CHUNK)
    ew2 = jnp.concatenate([edge_weight.astype(jnp.float32),
                           jnp.zeros((pad,), jnp.float32)]).reshape(-1, _CHUNK)
    zeros_blk = jnp.zeros((200, d), jnp.float32)
    parts = _sc_spmm(fea, src2, dst2, ew2, zeros_blk, nc)
    return _tc_matmul_bias(parts, W, b)


# speedup vs baseline: 1.1670x; 1.0748x over previous
import functools

import jax
import jax.numpy as jnp
from jax import lax
from jax.experimental import pallas as pl
from jax.experimental.pallas import tpu as pltpu
from jax.experimental.pallas import tpu_sc as plsc

_NC = 2
_NS = 16
_CHUNK = 112
_NBUF = 3
_G = 3
_LOOK = 2


def _sc_spmm(fea, src2, dst2, ew2, zeros_blk, nc):
    n, d = fea.shape
    piece = 200
    pieces = n // piece
    groups = d // 16
    nseg = nc // _G

    mesh = plsc.VectorSubcoreMesh(core_axis_name="c", subcore_axis_name="s")

    @functools.partial(
        pl.kernel,
        out_type=jax.ShapeDtypeStruct((_NC * n, d), jnp.float32),
        mesh=mesh,
        scratch_types=[
            pltpu.VMEM((2 * _G * _CHUNK,), jnp.int32),
            pltpu.VMEM((2 * _G * _CHUNK,), jnp.int32),
            pltpu.VMEM((2 * _G * _CHUNK,), jnp.float32),
            pltpu.VMEM((_NBUF, 8, _CHUNK), jnp.int32),
            pltpu.VMEM((_NBUF, _CHUNK, d), jnp.float32),
            pltpu.VMEM_SHARED((n, d), jnp.float32),
            [pltpu.SemaphoreType.DMA] * _NBUF,
            [pltpu.SemaphoreType.DMA] * _NBUF,
            [pltpu.SemaphoreType.DMA] * 2,
        ],
    )
    def spmm(fea_hbm, src_hbm, dst_hbm, ew_hbm, zero_hbm, out_hbm,
             src_v, dst_v, ew_v, dstc, rows_v, acc, gsems, ssems, isems):
        cid = lax.axis_index("c")
        sid = lax.axis_index("s")
        wid = sid * _NC + cid
        c0 = wid * nc
        sgl = _G * _CHUNK

        def fire_seg(seg, slot):
            off = (c0 + seg * _G) * _CHUNK
            pltpu.async_copy(src_hbm.at[pl.ds(off, sgl)],
                             src_v.at[pl.ds(slot * sgl, sgl)], isems[slot])
            pltpu.async_copy(dst_hbm.at[pl.ds(off, sgl)],
                             dst_v.at[pl.ds(slot * sgl, sgl)], isems[slot])
            pltpu.async_copy(ew_hbm.at[pl.ds(off, sgl)],
                             ew_v.at[pl.ds(slot * sgl, sgl)], isems[slot])

        def wait_seg(slot):
            pltpu.make_async_copy(src_hbm.at[pl.ds(0, sgl)],
                                  src_v.at[pl.ds(slot * sgl, sgl)],
                                  isems[slot]).wait()
            pltpu.make_async_copy(dst_hbm.at[pl.ds(0, sgl)],
                                  dst_v.at[pl.ds(slot * sgl, sgl)],
                                  isems[slot]).wait()
            pltpu.make_async_copy(ew_hbm.at[pl.ds(0, sgl)],
                                  ew_v.at[pl.ds(slot * sgl, sgl)],
                                  isems[slot]).wait()

        fire_seg(0, 0)
        wait_seg(0)
        fire_seg(1, 1)

        my_pieces = (pieces // _NS) + jnp.where(sid < (pieces % _NS), 1, 0)

        def init_body(t, c_):
            off = (sid + t * _NS) * piece
            pltpu.sync_copy(zero_hbm, acc.at[pl.ds(off, piece)])
            return c_

        lax.fori_loop(0, my_pieces, init_body, 0)
        plsc.subcore_barrier()

        def fire_gather(j, b):
            slot, pos = (j // _G) % 2, j % _G
            pltpu.async_copy(
                fea_hbm.at[src_v.at[pl.ds((slot * _G + pos) * _CHUNK, _CHUNK)]],
                rows_v.at[b], gsems[b])

        def wait_gather(b):
            pltpu.make_async_copy(
                fea_hbm.at[src_v.at[pl.ds(0, _CHUNK)]], rows_v.at[b],
                gsems[b]).wait()

        def fire_scatter(j, b):
            pltpu.async_copy(rows_v.at[b], acc.at[dstc.at[b, 0]],
                             ssems[b], add=True)

        def wait_scatter(b):
            pltpu.make_async_copy(rows_v.at[b], acc.at[dstc.at[b, 0]],
                                  ssems[b]).wait()

        def scale(j, b):
            slot, pos = (j // _G) % 2, j % _G

            def scale_body(s, c_):
                base = (slot * _G + pos) * _CHUNK + s * 16
                wv = ew_v[pl.ds(base, 16)]
                dstc[b, 0, pl.ds(s * 16, 16)] = dst_v[pl.ds(base, 16)]
                for lane in range(16):
                    w = wv[lane]
                    row = s * 16 + lane
                    for g in range(groups):
                        sl = pl.ds(g * 16, 16)
                        rows_v[b, row, sl] = rows_v[b, row, sl] * w
                return c_

            lax.fori_loop(0, _CHUNK // 16, scale_body, 0)

        for jj in range(_LOOK):
            fire_gather(jj, jj)

        def pipe_body(t, c_):
            seg = t
            for b in range(_NBUF):
                j = t * _NBUF + b

                wait_gather(b)
                scale(j, b)
                fire_scatter(j, b)

                @pl.when(j >= 1)
                def _():
                    wait_scatter((b + _NBUF - 1) % _NBUF)

                if b == 0:
                    refill = (seg >= 1) & (seg + 1 < nseg)
                    for slot in range(2):
                        @pl.when(refill & ((seg + 1) % 2 == slot))
                        def _(slot=slot):
                            fire_seg(seg + 1, slot)

                if b == _G - _LOOK:
                    for slot in range(2):
                        @pl.when((seg + 1 < nseg) & ((seg + 1) % 2 == slot))
                        def _(slot=slot):
                            wait_seg(slot)

                @pl.when(j + _LOOK < nc)
                def _():
                    fire_gather(j + _LOOK, (b + _LOOK) % _NBUF)

            return c_

        lax.fori_loop(0, nc // _NBUF, pipe_body, 0)
        wait_scatter((nc - 1) % _NBUF)
        plsc.subcore_barrier()

        def out_body(t, c_):
            off = (sid + t * _NS) * piece
            pltpu.sync_copy(acc.at[pl.ds(off, piece)],
                            out_hbm.at[pl.ds(cid * n + off, piece)])
            return c_

        lax.fori_loop(0, my_pieces, out_body, 0)

    return spmm(fea, src2, dst2, ew2, zeros_blk)


def _mm_body(p0_ref, p1_ref, w_ref, b_ref, o_ref):
    s = p0_ref[...] + p1_ref[...]
    o_ref[...] = jnp.dot(s, w_ref[...], preferred_element_type=jnp.float32) + b_ref[...]


def _tc_matmul_bias(parts, w, b):
    n2, d_in = parts.shape
    n = n2 // _NC
    d_out = w.shape[1]
    blk = 1000
    nb = n // blk
    return pl.pallas_call(
        _mm_body,
        grid=(nb,),
        in_specs=[
            pl.BlockSpec((blk, d_in), lambda i: (i, 0)),
            pl.BlockSpec((blk, d_in), lambda i: (i + nb, 0)),
            pl.BlockSpec((d_in, d_out), lambda i: (0, 0)),
            pl.BlockSpec((1, d_out), lambda i: (0, 0)),
        ],
        out_specs=pl.BlockSpec((blk, d_out), lambda i: (i, 0)),
        out_shape=jax.ShapeDtypeStruct((n, d_out), jnp.float32),
    )(parts, parts, w, b.reshape(1, d_out))


def kernel(fea, edge_index, edge_weight, W, b):
    src = edge_index[0].astype(jnp.int32)
    dst = edge_index[1].astype(jnp.int32)
    n, d = fea.shape
    e = src.shape[0]
    nw = _NC * _NS
    unit = _NBUF * _G * _CHUNK
    per_tile = -(-e // (nw * unit)) * unit
    e_pad = nw * per_tile
    nc = per_tile // _CHUNK
    pad = e_pad - e
    fill = (jnp.arange(pad, dtype=jnp.int32) * 7) % jnp.int32(n)
    src2 = jnp.concatenate([src, fill])
    dst2 = jnp.concatenate([dst, fill])
    ew2 = jnp.concatenate([edge_weight.astype(jnp.float32),
                           jnp.zeros((pad,), jnp.float32)])
    zeros_blk = jnp.zeros((200, d), jnp.float32)
    parts = _sc_spmm(fea, src2, dst2, ew2, zeros_blk, nc)
    return _tc_matmul_bias(parts, W, b)
